# baseline (speedup 1.0000x reference)
import jax
import jax.numpy as jnp
from jax import lax
from jax.experimental import pallas as pl
from jax.experimental.pallas import tpu as pltpu
from jax.experimental.pallas import tpu_sc as plsc

B, L, H = 16, 4096, 768
MAX_LABEL = 24
ROWS = 32
NC, NS = 2, 16
NW = NC * NS
TOK_W = (B * L) // NW
SC_CH = 64
NCH = TOK_W // SC_CH


def _sc_body(hs_hbm, lm_hbm, z_hbm, sums_hbm, cnts_hbm,
             acc_v, cacc_v, lm_v, rows_v, sem):
    c = lax.axis_index("c")
    s = lax.axis_index("s")
    wid = c * NS + s
    base = wid * TOK_W

    pltpu.sync_copy(z_hbm, acc_v)
    zero16 = jnp.zeros((16,), jnp.float32)
    cacc_v[pl.ds(0, 16)] = zero16
    cacc_v[pl.ds(16, 16)] = zero16
    pltpu.sync_copy(lm_hbm.at[pl.ds(base, TOK_W)], lm_v.at[pl.ds(0, TOK_W)])

    iota16 = lax.broadcasted_iota(jnp.int32, (16,), 0)
    ones16 = jnp.ones((16,), jnp.float32)

    def start_gather(i, buf):
        pltpu.async_copy(hs_hbm.at[pl.ds(base + i * SC_CH, SC_CH)],
                         rows_v.at[buf], sem)

    def drain(buf):
        pltpu.make_async_copy(hs_hbm.at[pl.ds(0, SC_CH)],
                              rows_v.at[buf], sem).wait()

    start_gather(0, 0)
    start_gather(1, 1)

    @pl.loop(0, NCH, step=2)
    def _chunks(i):
        for b in range(2):
            drain(b)
            chunk0 = i * SC_CH + b * SC_CH
            for g in range(SC_CH // 16):
                lab16 = lm_v[pl.ds(chunk0 + g * 16, 16)]
                plsc.addupdate_scatter(cacc_v, [lab16], ones16)

            @pl.loop(0, SC_CH)
            def _tok(t):
                rbase = lm_v[pl.ds(chunk0 + t, 16)][0] * H
                for j in range(H // 16):
                    vals = rows_v[b, t, pl.ds(j * 16, 16)]
                    plsc.addupdate(acc_v.at[pl.ds(rbase + j * 16, 16)], vals)

            @pl.when(i + b + 2 < NCH)
            def _():
                start_gather(i + b + 2, b)

    plsc.subcore_barrier()
    pltpu.sync_copy(acc_v, sums_hbm.at[wid])
    pltpu.sync_copy(cacc_v, cnts_hbm.at[wid])


def _sc_segment_sums(hs_flat, lm_flat):
    mesh = plsc.VectorSubcoreMesh(core_axis_name="c", subcore_axis_name="s")
    zeros_acc = jnp.zeros((ROWS * H,), jnp.float32)
    f = pl.kernel(
        _sc_body,
        out_type=[
            jax.ShapeDtypeStruct((NW, ROWS * H), jnp.float32),
            jax.ShapeDtypeStruct((NW, ROWS), jnp.float32),
        ],
        mesh=mesh,
        compiler_params=pltpu.CompilerParams(needs_layout_passes=False),
        scratch_types=[
            pltpu.VMEM((ROWS * H,), jnp.float32),
            pltpu.VMEM((ROWS,), jnp.float32),
            pltpu.VMEM((TOK_W + 16,), jnp.int32),
            pltpu.VMEM((2, SC_CH, H), jnp.float32),
            pltpu.SemaphoreType.DMA,
        ],
    )
    return f(hs_flat, lm_flat, zeros_acc)


def _fin_body(p0_ref, p1_ref, c0_ref, c1_ref, w_ref, b_ref, out_ref, valid_ref):
    cnt = c0_ref[...] + c1_ref[...]
    valid = (cnt > 0).astype(jnp.float32)
    mean = (p0_ref[...] + p1_ref[...]) / jnp.maximum(cnt, 1.0)
    proj = jax.lax.dot(mean, w_ref[...],
                       precision=jax.lax.Precision.DEFAULT,
                       preferred_element_type=jnp.float32) + b_ref[...]
    out_ref[...] = proj * valid
    valid_ref[...] = jnp.broadcast_to(valid, (B * ROWS, 128))


def _finish(p0, p1, c0, c1, W_label, b_label):
    return pl.pallas_call(
        _fin_body,
        out_shape=[
            jax.ShapeDtypeStruct((B * ROWS, H), jnp.float32),
            jax.ShapeDtypeStruct((B * ROWS, 128), jnp.float32),
        ],
    )(p0, p1, c0, c1, W_label, b_label.reshape(1, H))


def kernel(hidden_states, lmask, input_ids, attention_mask, W_label, b_label):
    hs_flat = hidden_states.reshape(B * L, H)
    lm_flat = lmask.astype(jnp.int32).reshape(B * L)
    sums, cnts = _sc_segment_sums(hs_flat, lm_flat)
    p = sums.reshape(B, 2, ROWS, H)
    cc = cnts.reshape(B, 2, ROWS, 1)
    out, valid = _finish(p[:, 0].reshape(B * ROWS, H),
                         p[:, 1].reshape(B * ROWS, H),
                         cc[:, 0].reshape(B * ROWS, 1),
                         cc[:, 1].reshape(B * ROWS, 1),
                         W_label, b_label)
    out3 = out.reshape(B, ROWS, H)[:, 1:MAX_LABEL + 1, :]
    aggregated = out3.reshape(B * MAX_LABEL, H)
    valid_mask = (valid.reshape(B, ROWS, 128)[:, 1:MAX_LABEL + 1, 0] > 0
                  ).reshape(B * MAX_LABEL)
    all_batch_ids = jnp.repeat(jnp.arange(B), MAX_LABEL)
    all_label_ids = jnp.tile(jnp.arange(1, MAX_LABEL + 1), B)
    return aggregated, all_batch_ids, all_label_ids, valid_mask


# speedup vs baseline: 2.4645x; 2.4645x over previous
import jax
import jax.numpy as jnp
from jax import lax
from jax.experimental import pallas as pl
from jax.experimental.pallas import tpu as pltpu
from jax.experimental.pallas import tpu_sc as plsc

B, L, H = 16, 4096, 768
MAX_LABEL = 24
ROWS = 32
NC, NS = 2, 16
NW = NC * NS
TOK_W = (B * L) // NW
SC_CH = 64
NCH = TOK_W // SC_CH


def _sc_body(hs_hbm, lm_hbm, z_hbm, sums_hbm, cnts_hbm,
             acc_v, cacc_v, lm_v, rows_v, sem):
    c = lax.axis_index("c")
    s = lax.axis_index("s")
    wid = c * NS + s
    base = wid * TOK_W

    pltpu.sync_copy(z_hbm, acc_v)
    zero16 = jnp.zeros((16,), jnp.float32)
    cacc_v[pl.ds(0, 16)] = zero16
    cacc_v[pl.ds(16, 16)] = zero16
    pltpu.sync_copy(lm_hbm.at[pl.ds(base, TOK_W)], lm_v.at[pl.ds(0, TOK_W)])

    iota16 = lax.broadcasted_iota(jnp.int32, (16,), 0)
    ones16 = jnp.ones((16,), jnp.float32)

    def start_gather(i, buf):
        pltpu.async_copy(hs_hbm.at[pl.ds(base + i * SC_CH, SC_CH)],
                         rows_v.at[buf], sem)

    def drain(buf):
        pltpu.make_async_copy(hs_hbm.at[pl.ds(0, SC_CH)],
                              rows_v.at[buf], sem).wait()

    start_gather(0, 0)
    start_gather(1, 1)

    @pl.loop(0, NCH, step=2)
    def _chunks(i):
        for b in range(2):
            drain(b)
            chunk0 = i * SC_CH + b * SC_CH
            for g in range(SC_CH // 16):
                lab16 = lm_v[pl.ds(chunk0 + g * 16, 16)]
                plsc.addupdate_scatter(cacc_v, [lab16], ones16)

            @plsc.parallel_loop(0, SC_CH)
            def _tok(t):
                rbase = lm_v[pl.ds(chunk0 + t, 16)][0] * H
                for j in range(H // 16):
                    vals = rows_v[b, t, pl.ds(j * 16, 16)]
                    plsc.addupdate(acc_v.at[pl.ds(rbase + j * 16, 16)], vals)

            @pl.when(i + b + 2 < NCH)
            def _():
                start_gather(i + b + 2, b)

    plsc.subcore_barrier()
    pltpu.sync_copy(acc_v, sums_hbm.at[wid])
    pltpu.sync_copy(cacc_v, cnts_hbm.at[wid])


def _sc_segment_sums(hs_flat, lm_flat):
    mesh = plsc.VectorSubcoreMesh(core_axis_name="c", subcore_axis_name="s")
    zeros_acc = jnp.zeros((ROWS * H,), jnp.float32)
    f = pl.kernel(
        _sc_body,
        out_type=[
            jax.ShapeDtypeStruct((NW, ROWS * H), jnp.float32),
            jax.ShapeDtypeStruct((NW, ROWS), jnp.float32),
        ],
        mesh=mesh,
        compiler_params=pltpu.CompilerParams(needs_layout_passes=False),
        scratch_types=[
            pltpu.VMEM((ROWS * H,), jnp.float32),
            pltpu.VMEM((ROWS,), jnp.float32),
            pltpu.VMEM((TOK_W + 16,), jnp.int32),
            pltpu.VMEM((2, SC_CH, H), jnp.float32),
            pltpu.SemaphoreType.DMA,
        ],
    )
    return f(hs_flat, lm_flat, zeros_acc)


def _fin_body(p0_ref, p1_ref, c0_ref, c1_ref, w_ref, b_ref, out_ref, valid_ref):
    cnt = c0_ref[...] + c1_ref[...]
    valid = (cnt > 0).astype(jnp.float32)
    mean = (p0_ref[...] + p1_ref[...]) / jnp.maximum(cnt, 1.0)
    proj = jax.lax.dot(mean, w_ref[...],
                       precision=jax.lax.Precision.DEFAULT,
                       preferred_element_type=jnp.float32) + b_ref[...]
    out_ref[...] = proj * valid
    valid_ref[...] = jnp.broadcast_to(valid, (B * ROWS, 128))


def _finish(p0, p1, c0, c1, W_label, b_label):
    return pl.pallas_call(
        _fin_body,
        out_shape=[
            jax.ShapeDtypeStruct((B * ROWS, H), jnp.float32),
            jax.ShapeDtypeStruct((B * ROWS, 128), jnp.float32),
        ],
    )(p0, p1, c0, c1, W_label, b_label.reshape(1, H))


def kernel(hidden_states, lmask, input_ids, attention_mask, W_label, b_label):
    hs_flat = hidden_states.reshape(B * L, H)
    lm_flat = lmask.astype(jnp.int32).reshape(B * L)
    sums, cnts = _sc_segment_sums(hs_flat, lm_flat)
    p = sums.reshape(B, 2, ROWS, H)
    cc = cnts.reshape(B, 2, ROWS, 1)
    out, valid = _finish(p[:, 0].reshape(B * ROWS, H),
                         p[:, 1].reshape(B * ROWS, H),
                         cc[:, 0].reshape(B * ROWS, 1),
                         cc[:, 1].reshape(B * ROWS, 1),
                         W_label, b_label)
    out3 = out.reshape(B, ROWS, H)[:, 1:MAX_LABEL + 1, :]
    aggregated = out3.reshape(B * MAX_LABEL, H)
    valid_mask = (valid.reshape(B, ROWS, 128)[:, 1:MAX_LABEL + 1, 0] > 0
                  ).reshape(B * MAX_LABEL)
    all_batch_ids = jnp.repeat(jnp.arange(B), MAX_LABEL)
    all_label_ids = jnp.tile(jnp.arange(1, MAX_LABEL + 1), B)
    return aggregated, all_batch_ids, all_label_ids, valid_mask
